# Initial kernel scaffold; baseline (speedup 1.0000x reference)
import jax
import jax.numpy as jnp
from jax.experimental import pallas as pl


def kernel(x, edge_index, edge_attr, candidate_idxs, batch, lin_in_w, lin_in_b, w1, b1, w2, b2, g1, be1, g2, be2, lin_out_w, lin_out_b):
    raise NotImplementedError("write your pallas kernel here")


# speedup vs baseline: 1.5730x; 1.5730x over previous
import functools

import jax
import jax.numpy as jnp
from jax import lax
from jax.experimental import pallas as pl
from jax.experimental.pallas import tpu as pltpu
from jax.experimental.pallas import tpu_sc as plsc

N = 50000
E = 800000
C = 5000
NG = 500
EMB = 32
L = 4

NW = 32
EW = 25088
E_PAD = NW * EW
CH = 128
BLK = 512
NBLK = EW // BLK
NCHW = EW // CH
TE = 2048
GRID_E = E_PAD // TE
TN = 2000
GRID_N = N // TN
NPS = N // 16
C_PAD = 5120
CW = C_PAD // NW
HT_W = 48
NG_PAD = 512
GT = 8
GRID_G = NG_PAD // GT
EPS = 1e-5

_mesh = plsc.VectorSubcoreMesh(core_axis_name="c", subcore_axis_name="s")


def _wid():
    return lax.axis_index("s") * 2 + lax.axis_index("c")


@functools.partial(
    pl.kernel, mesh=_mesh,
    compiler_params=pltpu.CompilerParams(use_tc_tiling_on_sc=False),
    out_type=[jax.ShapeDtypeStruct((E_PAD, EMB), jnp.float32),
              jax.ShapeDtypeStruct((E_PAD, EMB), jnp.float32)],
    scratch_types=[pltpu.VMEM((EW,), jnp.int32),
                   pltpu.VMEM((EW,), jnp.int32),
                   pltpu.VMEM((BLK, EMB), jnp.float32),
                   pltpu.VMEM((BLK, EMB), jnp.float32),
                   pltpu.SemaphoreType.DMA],
)
def _gather_edges(hi_hbm, hj_hbm, dst_hbm, src_hbm, ta_hbm, tb_hbm,
                  idxd, idxs, rowsA, rowsB, sem):
    w = _wid()
    base = w * EW
    pltpu.sync_copy(dst_hbm.at[pl.ds(base, EW)], idxd)
    pltpu.sync_copy(src_hbm.at[pl.ds(base, EW)], idxs)

    def body(bi, carry):
        bo = bi * BLK
        descs = []
        for j in range(BLK // CH):
            descs.append(pltpu.async_copy(
                hi_hbm.at[idxd.at[pl.ds(bo + j * CH, CH)]],
                rowsA.at[pl.ds(j * CH, CH)], sem))
            descs.append(pltpu.async_copy(
                hj_hbm.at[idxs.at[pl.ds(bo + j * CH, CH)]],
                rowsB.at[pl.ds(j * CH, CH)], sem))
        for d in descs:
            d.wait()
        pltpu.sync_copy(rowsA, ta_hbm.at[pl.ds(base + bo, BLK)])
        pltpu.sync_copy(rowsB, tb_hbm.at[pl.ds(base + bo, BLK)])
        return carry

    lax.fori_loop(0, NBLK, body, 0)


HEMB = EMB // 2
ES = E_PAD // 16
NCHS = ES // CH
NBLKS = ES // BLK


@functools.partial(
    pl.kernel, mesh=_mesh,
    compiler_params=pltpu.CompilerParams(use_tc_tiling_on_sc=False),
    out_type=jax.ShapeDtypeStruct((2, N, HEMB), jnp.float32),
    scratch_types=[pltpu.VMEM((NCHS, CH), jnp.int32),
                   pltpu.VMEM((BLK, HEMB), jnp.float32),
                   pltpu.VMEM((125, HEMB), jnp.float32),
                   pltpu.VMEM_SHARED((N, HEMB), jnp.float32),
                   pltpu.SemaphoreType.DMA],
)
def _scatter_agg(a2s_hbm, dst3_hbm, zrows_hbm, out_hbm,
                 idx2, upd, zbuf, table, sem):
    c = lax.axis_index("c")
    s = lax.axis_index("s")
    base = s * ES
    pltpu.sync_copy(zrows_hbm, zbuf)
    for k in range(25):
        pltpu.sync_copy(zbuf, table.at[pl.ds(s * NPS + k * 125, 125)])
    plsc.subcore_barrier()
    pltpu.sync_copy(dst3_hbm.at[s], idx2)

    def body(bi, carry):
        bo = bi * BLK
        pltpu.sync_copy(a2s_hbm.at[c, pl.ds(base + bo, BLK)], upd)
        descs = []
        for j in range(BLK // CH):
            descs.append(pltpu.async_copy(
                upd.at[pl.ds(j * CH, CH)],
                table.at[idx2.at[bi * (BLK // CH) + j]],
                sem, add=True))
        for d in descs:
            d.wait()
        return carry

    lax.fori_loop(0, NBLKS, body, 0)
    plsc.subcore_barrier()
    pltpu.sync_copy(table.at[pl.ds(s * NPS, NPS)],
                    out_hbm.at[c, pl.ds(s * NPS, NPS)])


@functools.partial(
    pl.kernel, mesh=_mesh,
    compiler_params=pltpu.CompilerParams(use_tc_tiling_on_sc=False),
    out_type=jax.ShapeDtypeStruct((C_PAD, HT_W), jnp.float32),
    scratch_types=[pltpu.VMEM((CW,), jnp.int32),
                   pltpu.VMEM((CW, HT_W), jnp.float32),
                   pltpu.SemaphoreType.DMA],
)
def _gather_cands(ht_hbm, cand_hbm, out_hbm, idxc, rows, sem):
    w = _wid()
    base = w * CW
    pltpu.sync_copy(cand_hbm.at[pl.ds(base, CW)], idxc)
    d1 = pltpu.async_copy(ht_hbm.at[idxc.at[pl.ds(0, CH)]],
                          rows.at[pl.ds(0, CH)], sem)
    d2 = pltpu.async_copy(ht_hbm.at[idxc.at[pl.ds(CH, CW - CH)]],
                          rows.at[pl.ds(CH, CW - CH)], sem)
    d1.wait()
    d2.wait()
    pltpu.sync_copy(rows, out_hbm.at[pl.ds(base, CW)])


def _tc_call(body, grid, in_specs, out_specs, out_shape, scratch=None):
    return pl.pallas_call(
        body, grid=grid, in_specs=in_specs, out_specs=out_specs,
        out_shape=out_shape, scratch_shapes=scratch or [])


def _h0_body(x_ref, w_ref, b_ref, o_ref):
    o_ref[...] = jnp.dot(x_ref[...], w_ref[...],
                         preferred_element_type=jnp.float32) + b_ref[0:1, :]


def _dense_body(h_ref, aA_ref, aB_ref, wi_ref, wj_ref, hn_ref, hi_ref, hj_ref):
    hn = h_ref[...] + jnp.concatenate([aA_ref[...], aB_ref[...]], axis=1)
    hn_ref[...] = hn
    hi_ref[...] = jnp.dot(hn, wi_ref[...], preferred_element_type=jnp.float32)
    hj_ref[...] = jnp.dot(hn, wj_ref[...], preferred_element_type=jnp.float32)


def _m1_of(tA, tB, ea, weT, b1_ref):
    c = jnp.dot(ea, weT, preferred_element_type=jnp.float32)
    return tA + tB + c + b1_ref[0:1, :]


def _rowmask(pid):
    gid = pid * TE + lax.broadcasted_iota(jnp.int32, (TE, 1), 0)
    return gid < E


def _finish_stats(acc_ref, g_ref, be_ref, st_ref):
    mean = acc_ref[0:1, :] / E
    var = acc_ref[1:2, :] / E - mean * mean
    sc = g_ref[0:1, :] / jnp.sqrt(var + EPS)
    sh = be_ref[0:1, :] - sc * mean
    st_ref[...] = jnp.concatenate(
        [sc, sh, jnp.zeros((6, EMB), jnp.float32)], axis=0)


def _stats1_body(tA_ref, tB_ref, ea_ref, weT_ref, b1_ref, g1_ref, be1_ref,
                 st_ref, acc_ref):
    pid = pl.program_id(0)

    @pl.when(pid == 0)
    def _():
        acc_ref[...] = jnp.zeros_like(acc_ref)

    m1 = _m1_of(tA_ref[...], tB_ref[...], ea_ref[...], weT_ref[...], b1_ref)
    m1 = jnp.where(_rowmask(pid), m1, 0.0)
    acc_ref[0:1, :] += jnp.sum(m1, axis=0, keepdims=True)
    acc_ref[1:2, :] += jnp.sum(m1 * m1, axis=0, keepdims=True)

    @pl.when(pid == GRID_E - 1)
    def _():
        _finish_stats(acc_ref, g1_ref, be1_ref, st_ref)


def _mlp_body(tA_ref, tB_ref, ea_ref, weT_ref, b1_ref, st1_ref, w2T_ref,
              b2_ref, g2_ref, be2_ref, m2_ref, st2_ref, acc_ref):
    pid = pl.program_id(0)

    @pl.when(pid == 0)
    def _():
        acc_ref[...] = jnp.zeros_like(acc_ref)

    m1 = _m1_of(tA_ref[...], tB_ref[...], ea_ref[...], weT_ref[...], b1_ref)
    a1 = jnp.maximum(m1 * st1_ref[0:1, :] + st1_ref[1:2, :], 0.0)
    m2 = jnp.dot(a1, w2T_ref[...], preferred_element_type=jnp.float32) \
        + b2_ref[0:1, :]
    m2_ref[...] = m2
    m2m = jnp.where(_rowmask(pid), m2, 0.0)
    acc_ref[0:1, :] += jnp.sum(m2m, axis=0, keepdims=True)
    acc_ref[1:2, :] += jnp.sum(m2m * m2m, axis=0, keepdims=True)

    @pl.when(pid == GRID_E - 1)
    def _():
        _finish_stats(acc_ref, g2_ref, be2_ref, st2_ref)


def _apply2_body(m2_ref, st2_ref, a2_ref):
    pid = pl.program_id(0)
    a2 = jnp.maximum(m2_ref[...] * st2_ref[0:1, :] + st2_ref[1:2, :], 0.0)
    a2 = jnp.where(_rowmask(pid), a2, 0.0)
    a2_ref[...] = jnp.stack([a2[:, 0:HEMB], a2[:, HEMB:EMB]], axis=0)


def _aug_body(h_ref, aA_ref, aB_ref, bt_ref, ht_ref):
    hn = h_ref[...] + jnp.concatenate([aA_ref[...], aB_ref[...]], axis=1)
    pad = jnp.zeros((TN, HT_W - EMB - 1), jnp.float32)
    ht_ref[...] = jnp.concatenate(
        [hn, bt_ref[...].astype(jnp.float32), pad], axis=1)


def _oht(seg, pid):
    gids = lax.broadcasted_iota(jnp.int32, (C_PAD, GT), 1) + pid * GT
    valid = lax.broadcasted_iota(jnp.int32, (C_PAD, 1), 0) < C
    return (seg.astype(jnp.int32) == gids) & valid


def _logits_of(hcb, lo_ref, lob_ref):
    return jnp.dot(hcb, lo_ref[...], preferred_element_type=jnp.float32)[
        :, 0:1] + lob_ref[0:1, 0:1]


def _seg_of(hcb):
    return hcb[:, EMB:EMB + 1]


def _mx_body(hcb_ref, lo_ref, lob_ref, lg_ref, mxc_ref, acc_ref):
    pid = pl.program_id(0)
    hcb = hcb_ref[...]
    lg = _logits_of(hcb, lo_ref, lob_ref)

    @pl.when(pid == 0)
    def _():
        acc_ref[...] = jnp.zeros_like(acc_ref)
        lg_ref[...] = lg

    oht = _oht(_seg_of(hcb), pid)
    masked = jnp.where(oht, lg, -1e30)
    mxrow = jnp.max(masked, axis=0, keepdims=True)
    acc_ref[...] += jnp.sum(jnp.where(oht, mxrow, 0.0), axis=1, keepdims=True)

    @pl.when(pid == GRID_G - 1)
    def _():
        mxc_ref[...] = acc_ref[...]


def _lse_body(hcb_ref, lg_ref, mxc_ref, out_ref, acc_ref):
    pid = pl.program_id(0)

    @pl.when(pid == 0)
    def _():
        acc_ref[...] = jnp.zeros_like(acc_ref)

    sh = lg_ref[...] - mxc_ref[...]
    valid = lax.broadcasted_iota(jnp.int32, (C_PAD, 1), 0) < C
    ex = jnp.where(valid, jnp.exp(sh), 0.0)
    oht = _oht(_seg_of(hcb_ref[...]), pid)
    srow = jnp.sum(jnp.where(oht, ex, 0.0), axis=0, keepdims=True)
    lserow = jnp.where(srow > 0.0, jnp.log(jnp.maximum(srow, 1e-37)), 0.0)
    acc_ref[...] += jnp.sum(jnp.where(oht, lserow, 0.0), axis=1, keepdims=True)

    @pl.when(pid == GRID_G - 1)
    def _():
        out_ref[...] = sh - acc_ref[...]


def kernel(x, edge_index, edge_attr, candidate_idxs, batch,
           lin_in_w, lin_in_b, w1, b1, w2, b2, g1, be1, g2, be2,
           lin_out_w, lin_out_b):
    f32 = jnp.float32
    src = edge_index[0]
    dst = edge_index[1]
    pad = E_PAD - E
    padidx = (jnp.arange(pad, dtype=jnp.int32) * 1031) % N
    dst_p = jnp.concatenate([dst, padidx])
    src_p = jnp.concatenate([src, padidx])
    dst3 = dst_p.reshape(16, NCHS, CH)
    ea8 = jnp.zeros((E_PAD, 8), f32).at[:E, 0:4].set(edge_attr)

    cpadidx = (jnp.arange(C_PAD - C, dtype=jnp.int32) * 997) % N
    cand_p = jnp.concatenate([candidate_idxs, cpadidx])

    x8 = jnp.zeros((N, 8), f32).at[:, 0:2].set(x)
    linT8 = jnp.zeros((8, EMB), f32).at[0:2, :].set(lin_in_w.T)
    b_in = jnp.broadcast_to(lin_in_b.reshape(1, EMB), (8, EMB))

    def row8(v):
        return jnp.broadcast_to(v.reshape(1, EMB), (8, EMB))

    wiT = [w1[l][:, 0:EMB].T for l in range(L)]
    wjT = [w1[l][:, EMB:2 * EMB].T for l in range(L)]
    weT8 = [jnp.zeros((8, EMB), f32).at[0:4, :].set(w1[l][:, 2 * EMB:].T)
            for l in range(L)]
    w2T = [w2[l].T for l in range(L)]
    b1r = [row8(b1[l]) for l in range(L)]
    b2r = [row8(b2[l]) for l in range(L)]
    g1r = [row8(g1[l]) for l in range(L)]
    be1r = [row8(be1[l]) for l in range(L)]
    g2r = [row8(g2[l]) for l in range(L)]
    be2r = [row8(be2[l]) for l in range(L)]

    h = _tc_call(
        _h0_body, (1,),
        [pl.BlockSpec((N, 8), lambda i: (0, 0)),
         pl.BlockSpec((8, EMB), lambda i: (0, 0)),
         pl.BlockSpec((8, EMB), lambda i: (0, 0))],
        pl.BlockSpec((N, EMB), lambda i: (0, 0)),
        jax.ShapeDtypeStruct((N, EMB), f32))(x8, linT8, b_in)

    zagg = jnp.zeros((N, HEMB), f32)
    aggA, aggB = zagg, zagg
    zrows = jnp.zeros((125, HEMB), f32)

    nspec = pl.BlockSpec((TN, EMB), lambda i: (i, 0))
    hspec = pl.BlockSpec((TN, HEMB), lambda i: (i, 0))
    espec = pl.BlockSpec((TE, EMB), lambda i: (i, 0))
    espec8 = pl.BlockSpec((TE, 8), lambda i: (i, 0))
    w32 = pl.BlockSpec((EMB, EMB), lambda i: (0, 0))
    p8 = pl.BlockSpec((8, EMB), lambda i: (0, 0))
    st_shape = jax.ShapeDtypeStruct((8, EMB), f32)
    acc2 = pltpu.VMEM((8, EMB), f32)

    for l in range(L):
        h, hi, hj = _tc_call(
            _dense_body, (GRID_N,),
            [nspec, hspec, hspec, w32, w32],
            [nspec, nspec, nspec],
            [jax.ShapeDtypeStruct((N, EMB), f32)] * 3,
        )(h, aggA, aggB, wiT[l], wjT[l])

        tA, tB = _gather_edges(hi, hj, dst_p, src_p)

        st1 = _tc_call(
            _stats1_body, (GRID_E,),
            [espec, espec, espec8, p8, p8, p8, p8],
            p8, st_shape, [acc2],
        )(tA, tB, ea8, weT8[l], b1r[l], g1r[l], be1r[l])

        m2, st2 = _tc_call(
            _mlp_body, (GRID_E,),
            [espec, espec, espec8, p8, p8, p8, w32, p8, p8, p8],
            [espec, p8],
            [jax.ShapeDtypeStruct((E_PAD, EMB), f32), st_shape], [acc2],
        )(tA, tB, ea8, weT8[l], b1r[l], st1, w2T[l], b2r[l], g2r[l], be2r[l])

        a2 = _tc_call(
            _apply2_body, (GRID_E,),
            [espec, p8],
            pl.BlockSpec((2, TE, HEMB), lambda i: (0, i, 0)),
            jax.ShapeDtypeStruct((2, E_PAD, HEMB), f32),
        )(m2, st2)

        agg = _scatter_agg(a2, dst3, zrows)
        aggA, aggB = agg[0], agg[1]

    bt = batch.reshape(N, 1)
    ht = _tc_call(
        _aug_body, (GRID_N,),
        [nspec, hspec, hspec, pl.BlockSpec((TN, 1), lambda i: (i, 0))],
        pl.BlockSpec((TN, HT_W), lambda i: (i, 0)),
        jax.ShapeDtypeStruct((N, HT_W), f32),
    )(h, aggA, aggB, bt)

    hcb = _gather_cands(ht, cand_p)

    lo8 = jnp.zeros((HT_W, 8), f32).at[0:EMB, 0:1].set(lin_out_w.T)
    lob = jnp.full((8, 8), lin_out_b, f32)
    cspec = pl.BlockSpec((C_PAD, HT_W), lambda i: (0, 0))
    c1spec = pl.BlockSpec((C_PAD, 1), lambda i: (0, 0))
    c1shape = jax.ShapeDtypeStruct((C_PAD, 1), f32)
    acc1 = pltpu.VMEM((C_PAD, 1), f32)

    lg, mxc = _tc_call(
        _mx_body, (GRID_G,),
        [cspec, pl.BlockSpec((HT_W, 8), lambda i: (0, 0)),
         pl.BlockSpec((8, 8), lambda i: (0, 0))],
        [c1spec, c1spec], [c1shape, c1shape], [acc1],
    )(hcb, lo8, lob)

    out = _tc_call(
        _lse_body, (GRID_G,),
        [cspec, c1spec, c1spec], c1spec, c1shape, [acc1],
    )(hcb, lg, mxc)

    return out[:C, 0]


# speedup vs baseline: 3.1977x; 2.0329x over previous
import functools

import jax
import jax.numpy as jnp
from jax import lax
from jax.experimental import pallas as pl
from jax.experimental.pallas import tpu as pltpu
from jax.experimental.pallas import tpu_sc as plsc

N = 50000
E = 800000
C = 5000
NG = 500
EMB = 32
L = 4

NW = 32
EW = 25088
E_PAD = NW * EW
CH = 128
BLK = 512
NBLK = EW // BLK
PR = E_PAD // 4
TP = 512
GRID_E = PR // TP
NR = N // 4
TNP = NR
GRID_N = 1
NPS = N // 16
C_PAD = 5120
CW = C_PAD // NW
NG_PAD = 512
GT = 8
GRID_G = NG_PAD // GT
EPS = 1e-5

HEMB = EMB // 2
ES = E_PAD // 16
NCHS = ES // CH
NBLKS = ES // BLK

_mesh = plsc.VectorSubcoreMesh(core_axis_name="c", subcore_axis_name="s")


def _wid():
    return lax.axis_index("s") * 2 + lax.axis_index("c")


@functools.partial(
    pl.kernel, mesh=_mesh,
    compiler_params=pltpu.CompilerParams(use_tc_tiling_on_sc=False),
    out_type=[jax.ShapeDtypeStruct((E_PAD, EMB), jnp.float32),
              jax.ShapeDtypeStruct((E_PAD, EMB), jnp.float32)],
    scratch_types=[pltpu.VMEM((EW,), jnp.int32),
                   pltpu.VMEM((EW,), jnp.int32),
                   pltpu.VMEM((BLK, EMB), jnp.float32),
                   pltpu.VMEM((BLK, EMB), jnp.float32),
                   pltpu.SemaphoreType.DMA],
)
def _gather_edges(hi_hbm, hj_hbm, dst_hbm, src_hbm, ta_hbm, tb_hbm,
                  idxd, idxs, rowsA, rowsB, sem):
    w = _wid()
    base = w * EW
    pltpu.sync_copy(dst_hbm.at[pl.ds(base, EW)], idxd)
    pltpu.sync_copy(src_hbm.at[pl.ds(base, EW)], idxs)

    def body(bi, carry):
        bo = bi * BLK
        descs = []
        for j in range(BLK // CH):
            descs.append(pltpu.async_copy(
                hi_hbm.at[idxd.at[pl.ds(bo + j * CH, CH)]],
                rowsA.at[pl.ds(j * CH, CH)], sem))
            descs.append(pltpu.async_copy(
                hj_hbm.at[idxs.at[pl.ds(bo + j * CH, CH)]],
                rowsB.at[pl.ds(j * CH, CH)], sem))
        for d in descs:
            d.wait()
        pltpu.sync_copy(rowsA, ta_hbm.at[pl.ds(base + bo, BLK)])
        pltpu.sync_copy(rowsB, tb_hbm.at[pl.ds(base + bo, BLK)])
        return carry

    lax.fori_loop(0, NBLK, body, 0)


@functools.partial(
    pl.kernel, mesh=_mesh,
    compiler_params=pltpu.CompilerParams(use_tc_tiling_on_sc=False),
    out_type=jax.ShapeDtypeStruct((N, EMB), jnp.float32),
    scratch_types=[pltpu.VMEM((NCHS, CH), jnp.int32),
                   pltpu.VMEM((BLK, HEMB), jnp.float32),
                   pltpu.VMEM((125, HEMB), jnp.float32),
                   pltpu.VMEM_SHARED((N, HEMB), jnp.float32),
                   pltpu.SemaphoreType.DMA],
)
def _scatter_agg(a2_hbm, dst3_hbm, zrows_hbm, out_hbm,
                 idx2, upd, zbuf, table, sem):
    c = lax.axis_index("c")
    s = lax.axis_index("s")
    base = s * ES
    pltpu.sync_copy(zrows_hbm, zbuf)
    for k in range(25):
        pltpu.sync_copy(zbuf, table.at[pl.ds(s * NPS + k * 125, 125)])
    plsc.subcore_barrier()
    pltpu.sync_copy(dst3_hbm.at[s], idx2)

    def run(coff):
        def body(bi, carry):
            bo = bi * BLK
            pltpu.sync_copy(
                a2_hbm.at[pl.ds(base + bo, BLK), pl.ds(coff, HEMB)], upd)
            descs = []
            for j in range(BLK // CH):
                descs.append(pltpu.async_copy(
                    upd.at[pl.ds(j * CH, CH)],
                    table.at[idx2.at[bi * (BLK // CH) + j]],
                    sem, add=True))
            for d in descs:
                d.wait()
            return carry

        lax.fori_loop(0, NBLKS, body, 0)
        plsc.subcore_barrier()
        pltpu.sync_copy(table.at[pl.ds(s * NPS, NPS)],
                        out_hbm.at[pl.ds(s * NPS, NPS), pl.ds(coff, HEMB)])

    @pl.when(c == 0)
    def _():
        run(0)

    @pl.when(c == 1)
    def _():
        run(HEMB)


@functools.partial(
    pl.kernel, mesh=_mesh,
    compiler_params=pltpu.CompilerParams(use_tc_tiling_on_sc=False),
    out_type=[jax.ShapeDtypeStruct((C_PAD, EMB), jnp.float32),
              jax.ShapeDtypeStruct((C_PAD,), jnp.int32)],
    scratch_types=[pltpu.VMEM((CW,), jnp.int32),
                   pltpu.VMEM((CW, EMB), jnp.float32),
                   pltpu.VMEM((CW,), jnp.int32),
                   pltpu.SemaphoreType.DMA],
)
def _gather_cands(h32_hbm, bat_hbm, cand_hbm, outr_hbm, outs_hbm,
                  idxc, rows, segv, sem):
    w = _wid()
    base = w * CW
    pltpu.sync_copy(cand_hbm.at[pl.ds(base, CW)], idxc)
    descs = [
        pltpu.async_copy(h32_hbm.at[idxc.at[pl.ds(0, CH)]],
                         rows.at[pl.ds(0, CH)], sem),
        pltpu.async_copy(h32_hbm.at[idxc.at[pl.ds(CH, CW - CH)]],
                         rows.at[pl.ds(CH, CW - CH)], sem),
        pltpu.async_copy(bat_hbm.at[idxc.at[pl.ds(0, CH)]],
                         segv.at[pl.ds(0, CH)], sem),
        pltpu.async_copy(bat_hbm.at[idxc.at[pl.ds(CH, CW - CH)]],
                         segv.at[pl.ds(CH, CW - CH)], sem),
    ]
    for d in descs:
        d.wait()
    pltpu.sync_copy(rows, outr_hbm.at[pl.ds(base, CW)])
    pltpu.sync_copy(segv, outs_hbm.at[pl.ds(base, CW)])


def _tc_call(body, grid, in_specs, out_specs, out_shape, scratch=None):
    return pl.pallas_call(
        body, grid=grid, in_specs=in_specs, out_specs=out_specs,
        out_shape=out_shape, scratch_shapes=scratch or [])


def _h0_body(x_ref, w_ref, b_ref, o_ref):
    o_ref[...] = jnp.dot(x_ref[...], w_ref[...],
                         preferred_element_type=jnp.float32) + b_ref[0:1, :]


def _dense_body(h_ref, agg_ref, wi_ref, wj_ref, hn_ref, hi_ref, hj_ref):
    hn = h_ref[...] + agg_ref[...]
    hn_ref[...] = hn
    hi_ref[...] = jnp.dot(hn, wi_ref[...], preferred_element_type=jnp.float32)
    hj_ref[...] = jnp.dot(hn, wj_ref[...], preferred_element_type=jnp.float32)


def _hfin_body(h_ref, agg_ref, hn_ref):
    hn_ref[...] = h_ref[...] + agg_ref[...]


def _m1_of(tA, tB, ea, weBD, b1_ref):
    c = jnp.dot(ea, weBD, preferred_element_type=jnp.float32)
    return tA + tB + c + b1_ref[0:1, :]


def _pmask(pid):
    er = pid * TP + lax.broadcasted_iota(jnp.int32, (TP, 128), 0)
    k = lax.broadcasted_iota(jnp.int32, (TP, 128), 1) // EMB
    return (er * 4 + k) < E


def _fold4(v):
    return v[:, 0:32] + v[:, 32:64] + v[:, 64:96] + v[:, 96:128]


def _finish_stats(acc_ref, g_ref, be_ref, st_ref):
    mean = _fold4(acc_ref[0:1, :]) / E
    var = _fold4(acc_ref[1:2, :]) / E - mean * mean
    rs = 1.0 / jnp.sqrt(var + EPS)
    rst = jnp.concatenate([rs] * 4, axis=1)
    mt = jnp.concatenate([mean] * 4, axis=1)
    scale = g_ref[0:1, :] * rst
    shift = be_ref[0:1, :] - scale * mt
    st_ref[...] = jnp.concatenate(
        [scale, shift, jnp.zeros((6, 128), jnp.float32)], axis=0)


def _stats1_body(tA_ref, tB_ref, ea_ref, weBD_ref, b1_ref, g1_ref, be1_ref,
                 st_ref, acc_ref):
    pid = pl.program_id(0)

    @pl.when(pid == 0)
    def _():
        acc_ref[...] = jnp.zeros_like(acc_ref)

    m1 = _m1_of(tA_ref[...], tB_ref[...], ea_ref[...], weBD_ref[...], b1_ref)
    m1 = jnp.where(_pmask(pid), m1, 0.0)
    acc_ref[0:1, :] += jnp.sum(m1, axis=0, keepdims=True)
    acc_ref[1:2, :] += jnp.sum(m1 * m1, axis=0, keepdims=True)

    @pl.when(pid == GRID_E - 1)
    def _():
        _finish_stats(acc_ref, g1_ref, be1_ref, st_ref)


def _stats2_body(tA_ref, tB_ref, ea_ref, weBD_ref, b1_ref, st1_ref, w2BD_ref,
                 b2_ref, g2_ref, be2_ref, st_ref, acc_ref):
    pid = pl.program_id(0)

    @pl.when(pid == 0)
    def _():
        acc_ref[...] = jnp.zeros_like(acc_ref)

    m1 = _m1_of(tA_ref[...], tB_ref[...], ea_ref[...], weBD_ref[...], b1_ref)
    a1 = jnp.maximum(m1 * st1_ref[0:1, :] + st1_ref[1:2, :], 0.0)
    m2 = jnp.dot(a1, w2BD_ref[...], preferred_element_type=jnp.float32) \
        + b2_ref[0:1, :]
    m2 = jnp.where(_pmask(pid), m2, 0.0)
    acc_ref[0:1, :] += jnp.sum(m2, axis=0, keepdims=True)
    acc_ref[1:2, :] += jnp.sum(m2 * m2, axis=0, keepdims=True)

    @pl.when(pid == GRID_E - 1)
    def _():
        _finish_stats(acc_ref, g2_ref, be2_ref, st_ref)


def _apply2_body(tA_ref, tB_ref, ea_ref, weBD_ref, b1_ref, st1_ref, w2BD_ref,
                 b2_ref, st2_ref, a2_ref):
    pid = pl.program_id(0)
    m1 = _m1_of(tA_ref[...], tB_ref[...], ea_ref[...], weBD_ref[...], b1_ref)
    a1 = jnp.maximum(m1 * st1_ref[0:1, :] + st1_ref[1:2, :], 0.0)
    m2 = jnp.dot(a1, w2BD_ref[...], preferred_element_type=jnp.float32) \
        + b2_ref[0:1, :]
    a2 = jnp.maximum(m2 * st2_ref[0:1, :] + st2_ref[1:2, :], 0.0)
    a2_ref[...] = jnp.where(_pmask(pid), a2, 0.0)


def _oht(seg, pid):
    gids = lax.broadcasted_iota(jnp.int32, (C_PAD, GT), 1) + pid * GT
    valid = lax.broadcasted_iota(jnp.int32, (C_PAD, 1), 0) < C
    return (seg == gids) & valid


def _mx_body(hc_ref, seg_ref, lo_ref, lob_ref, lg_ref, mxc_ref, acc_ref):
    pid = pl.program_id(0)
    lg = jnp.dot(hc_ref[...], lo_ref[...],
                 preferred_element_type=jnp.float32)[:, 0:1] \
        + lob_ref[0:1, 0:1]

    @pl.when(pid == 0)
    def _():
        acc_ref[...] = jnp.zeros_like(acc_ref)
        lg_ref[...] = lg

    oht = _oht(seg_ref[...], pid)
    masked = jnp.where(oht, lg, -1e30)
    mxrow = jnp.max(masked, axis=0, keepdims=True)
    acc_ref[...] += jnp.sum(jnp.where(oht, mxrow, 0.0), axis=1, keepdims=True)

    @pl.when(pid == GRID_G - 1)
    def _():
        mxc_ref[...] = acc_ref[...]


def _lse_body(seg_ref, lg_ref, mxc_ref, out_ref, acc_ref):
    pid = pl.program_id(0)

    @pl.when(pid == 0)
    def _():
        acc_ref[...] = jnp.zeros_like(acc_ref)

    sh = lg_ref[...] - mxc_ref[...]
    valid = lax.broadcasted_iota(jnp.int32, (C_PAD, 1), 0) < C
    ex = jnp.where(valid, jnp.exp(sh), 0.0)
    oht = _oht(seg_ref[...], pid)
    srow = jnp.sum(jnp.where(oht, ex, 0.0), axis=0, keepdims=True)
    lserow = jnp.where(srow > 0.0, jnp.log(jnp.maximum(srow, 1e-37)), 0.0)
    acc_ref[...] += jnp.sum(jnp.where(oht, lserow, 0.0), axis=1, keepdims=True)

    @pl.when(pid == GRID_G - 1)
    def _():
        out_ref[...] = sh - acc_ref[...]


def kernel(x, edge_index, edge_attr, candidate_idxs, batch,
           lin_in_w, lin_in_b, w1, b1, w2, b2, g1, be1, g2, be2,
           lin_out_w, lin_out_b):
    f32 = jnp.float32
    eye4 = jnp.eye(4, dtype=f32)

    def bd(m32):
        return jnp.kron(eye4, m32)

    def t8(v):
        return jnp.broadcast_to(jnp.tile(v, 4).reshape(1, 128), (8, 128))

    src = edge_index[0]
    dst = edge_index[1]
    pad = E_PAD - E
    padidx = (jnp.arange(pad, dtype=jnp.int32) * 1031) % N
    dst_p = jnp.concatenate([dst, padidx])
    src_p = jnp.concatenate([src, padidx])
    dst3 = dst_p.reshape(16, NCHS, CH)
    eaP = jnp.zeros((E_PAD, EMB), f32).at[:E, 0:4].set(edge_attr) \
        .reshape(PR, 128)

    cpadidx = (jnp.arange(C_PAD - C, dtype=jnp.int32) * 997) % N
    cand_p = jnp.concatenate([candidate_idxs, cpadidx])

    xP = jnp.zeros((N, EMB), f32).at[:, 0:2].set(x).reshape(NR, 128)
    lin32 = jnp.zeros((EMB, EMB), f32).at[0:2, :].set(lin_in_w.T)
    linBD = bd(lin32)
    b_in = t8(lin_in_b)

    wiBD = [bd(w1[l][:, 0:EMB].T) for l in range(L)]
    wjBD = [bd(w1[l][:, EMB:2 * EMB].T) for l in range(L)]
    weBD = [bd(jnp.zeros((EMB, EMB), f32).at[0:4, :].set(w1[l][:, 2 * EMB:].T))
            for l in range(L)]
    w2BD = [bd(w2[l].T) for l in range(L)]
    b1r = [t8(b1[l]) for l in range(L)]
    b2r = [t8(b2[l]) for l in range(L)]
    g1r = [t8(g1[l]) for l in range(L)]
    be1r = [t8(be1[l]) for l in range(L)]
    g2r = [t8(g2[l]) for l in range(L)]
    be2r = [t8(be2[l]) for l in range(L)]

    p128 = pl.BlockSpec((8, 128), lambda i: (0, 0))
    w128 = pl.BlockSpec((128, 128), lambda i: (0, 0))
    nspec = pl.BlockSpec((TNP, 128), lambda i: (i, 0))
    espec = pl.BlockSpec((TP, 128), lambda i: (i, 0))
    st_shape = jax.ShapeDtypeStruct((8, 128), f32)
    acc2 = pltpu.VMEM((8, 128), f32)

    hP = _tc_call(
        _h0_body, (1,),
        [pl.BlockSpec((NR, 128), lambda i: (0, 0)), w128, p128],
        pl.BlockSpec((NR, 128), lambda i: (0, 0)),
        jax.ShapeDtypeStruct((NR, 128), f32))(xP, linBD, b_in)

    aggP = jnp.zeros((NR, 128), f32)
    zrows = jnp.zeros((125, HEMB), f32)

    for l in range(L):
        hP, hiP, hjP = _tc_call(
            _dense_body, (GRID_N,),
            [nspec, nspec, w128, w128],
            [nspec, nspec, nspec],
            [jax.ShapeDtypeStruct((NR, 128), f32)] * 3,
        )(hP, aggP, wiBD[l], wjBD[l])

        tA, tB = _gather_edges(hiP.reshape(N, EMB), hjP.reshape(N, EMB),
                               dst_p, src_p)
        tAP = tA.reshape(PR, 128)
        tBP = tB.reshape(PR, 128)

        st1 = _tc_call(
            _stats1_body, (GRID_E,),
            [espec, espec, espec, w128, p128, p128, p128],
            p128, st_shape, [acc2],
        )(tAP, tBP, eaP, weBD[l], b1r[l], g1r[l], be1r[l])

        st2 = _tc_call(
            _stats2_body, (GRID_E,),
            [espec, espec, espec, w128, p128, p128, w128, p128, p128, p128],
            p128, st_shape, [acc2],
        )(tAP, tBP, eaP, weBD[l], b1r[l], st1, w2BD[l], b2r[l], g2r[l],
          be2r[l])

        a2P = _tc_call(
            _apply2_body, (GRID_E,),
            [espec, espec, espec, w128, p128, p128, w128, p128, p128],
            espec, jax.ShapeDtypeStruct((PR, 128), f32),
        )(tAP, tBP, eaP, weBD[l], b1r[l], st1, w2BD[l], b2r[l], st2)

        agg = _scatter_agg(a2P.reshape(E_PAD, EMB), dst3, zrows)
        aggP = agg.reshape(NR, 128)

    hfinP = _tc_call(
        _hfin_body, (GRID_N,),
        [nspec, nspec], nspec,
        jax.ShapeDtypeStruct((NR, 128), f32))(hP, aggP)

    hcb, segc = _gather_cands(hfinP.reshape(N, EMB), batch, cand_p)
    seg2 = segc.reshape(C_PAD, 1)

    lo8 = jnp.zeros((EMB, 8), f32).at[:, 0:1].set(lin_out_w.T)
    lob = jnp.full((8, 8), lin_out_b, f32)
    cspec = pl.BlockSpec((C_PAD, EMB), lambda i: (0, 0))
    s1spec = pl.BlockSpec((C_PAD, 1), lambda i: (0, 0))
    c1shape = jax.ShapeDtypeStruct((C_PAD, 1), f32)
    acc1 = pltpu.VMEM((C_PAD, 1), f32)

    lg, mxc = _tc_call(
        _mx_body, (GRID_G,),
        [cspec, s1spec, pl.BlockSpec((EMB, 8), lambda i: (0, 0)),
         pl.BlockSpec((8, 8), lambda i: (0, 0))],
        [s1spec, s1spec], [c1shape, c1shape], [acc1],
    )(hcb, seg2, lo8, lob)

    out = _tc_call(
        _lse_body, (GRID_G,),
        [s1spec, s1spec, s1spec], s1spec, c1shape, [acc1],
    )(seg2, lg, mxc)

    return out[:C, 0]


# speedup vs baseline: 3.2729x; 1.0235x over previous
import functools

import jax
import jax.numpy as jnp
from jax import lax
from jax.experimental import pallas as pl
from jax.experimental.pallas import tpu as pltpu
from jax.experimental.pallas import tpu_sc as plsc

N = 50000
E = 800000
C = 5000
NG = 500
EMB = 32
L = 4

NW = 32
EW = 25088
E_PAD = NW * EW
CH = 128
BLK = 512
NBLK = EW // BLK
PR = E_PAD // 4
TP = 512
GRID_E = PR // TP
NR = N // 4
TNP = NR
GRID_N = 1
NPS = N // 16
C_PAD = 5120
CW = C_PAD // NW
NG_PAD = 512
GT = 8
GRID_G = NG_PAD // GT
EPS = 1e-5

HEMB = EMB // 2
ES = E_PAD // 16
NCHS = ES // CH
NBLKS = ES // BLK

_mesh = plsc.VectorSubcoreMesh(core_axis_name="c", subcore_axis_name="s")


def _wid():
    return lax.axis_index("s") * 2 + lax.axis_index("c")


@functools.partial(
    pl.kernel, mesh=_mesh,
    compiler_params=pltpu.CompilerParams(use_tc_tiling_on_sc=False),
    out_type=[jax.ShapeDtypeStruct((E_PAD, EMB), jnp.float32),
              jax.ShapeDtypeStruct((E_PAD, EMB), jnp.float32)],
    scratch_types=[pltpu.VMEM((EW,), jnp.int32),
                   pltpu.VMEM((EW,), jnp.int32),
                   pltpu.VMEM((BLK, EMB), jnp.float32),
                   pltpu.VMEM((BLK, EMB), jnp.float32),
                   pltpu.SemaphoreType.DMA],
)
def _gather_edges(hi_hbm, hj_hbm, dst_hbm, src_hbm, ta_hbm, tb_hbm,
                  idxd, idxs, rowsA, rowsB, sem):
    w = _wid()
    base = w * EW
    pltpu.sync_copy(dst_hbm.at[pl.ds(base, EW)], idxd)
    pltpu.sync_copy(src_hbm.at[pl.ds(base, EW)], idxs)

    def body(bi, carry):
        bo = bi * BLK
        descs = []
        for j in range(BLK // CH):
            descs.append(pltpu.async_copy(
                hi_hbm.at[idxd.at[pl.ds(bo + j * CH, CH)]],
                rowsA.at[pl.ds(j * CH, CH)], sem))
            descs.append(pltpu.async_copy(
                hj_hbm.at[idxs.at[pl.ds(bo + j * CH, CH)]],
                rowsB.at[pl.ds(j * CH, CH)], sem))
        for d in descs:
            d.wait()
        pltpu.sync_copy(rowsA, ta_hbm.at[pl.ds(base + bo, BLK)])
        pltpu.sync_copy(rowsB, tb_hbm.at[pl.ds(base + bo, BLK)])
        return carry

    lax.fori_loop(0, NBLK, body, 0)


@functools.partial(
    pl.kernel, mesh=_mesh,
    compiler_params=pltpu.CompilerParams(use_tc_tiling_on_sc=False),
    out_type=jax.ShapeDtypeStruct((N, EMB), jnp.float32),
    scratch_types=[pltpu.VMEM((NCHS, CH), jnp.int32),
                   pltpu.VMEM((BLK, HEMB), jnp.float32),
                   pltpu.VMEM((125, HEMB), jnp.float32),
                   pltpu.VMEM_SHARED((N, HEMB), jnp.float32),
                   pltpu.SemaphoreType.DMA],
)
def _scatter_agg(a2_hbm, dst3_hbm, zrows_hbm, out_hbm,
                 idx2, upd, zbuf, table, sem):
    c = lax.axis_index("c")
    s = lax.axis_index("s")
    base = s * ES
    pltpu.sync_copy(zrows_hbm, zbuf)
    for k in range(25):
        pltpu.sync_copy(zbuf, table.at[pl.ds(s * NPS + k * 125, 125)])
    plsc.subcore_barrier()
    pltpu.sync_copy(dst3_hbm.at[s], idx2)

    def run(coff):
        def body(bi, carry):
            bo = bi * BLK
            pltpu.sync_copy(
                a2_hbm.at[pl.ds(base + bo, BLK), pl.ds(coff, HEMB)], upd)
            descs = []
            for j in range(BLK // CH):
                descs.append(pltpu.async_copy(
                    upd.at[pl.ds(j * CH, CH)],
                    table.at[idx2.at[bi * (BLK // CH) + j]],
                    sem, add=True))
            for d in descs:
                d.wait()
            return carry

        lax.fori_loop(0, NBLKS, body, 0)
        plsc.subcore_barrier()
        pltpu.sync_copy(table.at[pl.ds(s * NPS, NPS)],
                        out_hbm.at[pl.ds(s * NPS, NPS), pl.ds(coff, HEMB)])

    @pl.when(c == 0)
    def _():
        run(0)

    @pl.when(c == 1)
    def _():
        run(HEMB)


@functools.partial(
    pl.kernel, mesh=_mesh,
    compiler_params=pltpu.CompilerParams(use_tc_tiling_on_sc=False),
    out_type=[jax.ShapeDtypeStruct((C_PAD, EMB), jnp.float32),
              jax.ShapeDtypeStruct((C_PAD,), jnp.int32)],
    scratch_types=[pltpu.VMEM((CW,), jnp.int32),
                   pltpu.VMEM((CW, EMB), jnp.float32),
                   pltpu.VMEM((CW,), jnp.int32),
                   pltpu.SemaphoreType.DMA],
)
def _gather_cands(h32_hbm, bat_hbm, cand_hbm, outr_hbm, outs_hbm,
                  idxc, rows, segv, sem):
    w = _wid()
    base = w * CW
    pltpu.sync_copy(cand_hbm.at[pl.ds(base, CW)], idxc)
    descs = [
        pltpu.async_copy(h32_hbm.at[idxc.at[pl.ds(0, CH)]],
                         rows.at[pl.ds(0, CH)], sem),
        pltpu.async_copy(h32_hbm.at[idxc.at[pl.ds(CH, CW - CH)]],
                         rows.at[pl.ds(CH, CW - CH)], sem),
        pltpu.async_copy(bat_hbm.at[idxc.at[pl.ds(0, CH)]],
                         segv.at[pl.ds(0, CH)], sem),
        pltpu.async_copy(bat_hbm.at[idxc.at[pl.ds(CH, CW - CH)]],
                         segv.at[pl.ds(CH, CW - CH)], sem),
    ]
    for d in descs:
        d.wait()
    pltpu.sync_copy(rows, outr_hbm.at[pl.ds(base, CW)])
    pltpu.sync_copy(segv, outs_hbm.at[pl.ds(base, CW)])


def _tc_call(body, grid, in_specs, out_specs, out_shape, scratch=None):
    return pl.pallas_call(
        body, grid=grid, in_specs=in_specs, out_specs=out_specs,
        out_shape=out_shape, scratch_shapes=scratch or [])


EAT = 1280
EAO = EAT // 4
GRID_EA = E // EAT


def _eapack_body(ea_ref, s_ref, t16_ref, o_ref):
    x = ea_ref[...]
    acc = jnp.zeros((EAO, 128), jnp.float32)
    for k in range(4):
        xk = jnp.dot(s_ref[k * EAO:(k + 1) * EAO, :], x,
                     preferred_element_type=jnp.float32)
        acc = acc + jnp.dot(xk, t16_ref[4 * k:4 * k + 4, :],
                            preferred_element_type=jnp.float32)
    o_ref[...] = acc


def _h0_body(x_ref, w_ref, b_ref, o_ref):
    o_ref[...] = jnp.dot(x_ref[...], w_ref[...],
                         preferred_element_type=jnp.float32) + b_ref[0:1, :]


def _dense_body(h_ref, agg_ref, wi_ref, wj_ref, hn_ref, hi_ref, hj_ref):
    hn = h_ref[...] + agg_ref[...]
    hn_ref[...] = hn
    hi_ref[...] = jnp.dot(hn, wi_ref[...], preferred_element_type=jnp.float32)
    hj_ref[...] = jnp.dot(hn, wj_ref[...], preferred_element_type=jnp.float32)


def _hfin_body(h_ref, agg_ref, hn_ref):
    hn_ref[...] = h_ref[...] + agg_ref[...]


def _m1_of(tA, tB, ea, weBD, b1_ref):
    c = jnp.dot(ea, weBD, preferred_element_type=jnp.float32)
    return tA + tB + c + b1_ref[0:1, :]


def _pmask(pid):
    er = pid * TP + lax.broadcasted_iota(jnp.int32, (TP, 128), 0)
    k = lax.broadcasted_iota(jnp.int32, (TP, 128), 1) // EMB
    return (er * 4 + k) < E


def _fold4(v):
    return v[:, 0:32] + v[:, 32:64] + v[:, 64:96] + v[:, 96:128]


def _finish_stats(acc_ref, g_ref, be_ref, st_ref):
    mean = _fold4(acc_ref[0:1, :]) / E
    var = _fold4(acc_ref[1:2, :]) / E - mean * mean
    rs = 1.0 / jnp.sqrt(var + EPS)
    rst = jnp.concatenate([rs] * 4, axis=1)
    mt = jnp.concatenate([mean] * 4, axis=1)
    scale = g_ref[0:1, :] * rst
    shift = be_ref[0:1, :] - scale * mt
    st_ref[...] = jnp.concatenate(
        [scale, shift, jnp.zeros((6, 128), jnp.float32)], axis=0)


def _stats1_body(tA_ref, tB_ref, ea_ref, weBD_ref, b1_ref, g1_ref, be1_ref,
                 st_ref, acc_ref):
    pid = pl.program_id(0)

    @pl.when(pid == 0)
    def _():
        acc_ref[...] = jnp.zeros_like(acc_ref)

    m1 = _m1_of(tA_ref[...], tB_ref[...], ea_ref[...], weBD_ref[...], b1_ref)
    m1 = jnp.where(_pmask(pid), m1, 0.0)
    acc_ref[0:1, :] += jnp.sum(m1, axis=0, keepdims=True)
    acc_ref[1:2, :] += jnp.sum(m1 * m1, axis=0, keepdims=True)

    @pl.when(pid == GRID_E - 1)
    def _():
        _finish_stats(acc_ref, g1_ref, be1_ref, st_ref)


def _stats2_body(tA_ref, tB_ref, ea_ref, weBD_ref, b1_ref, st1_ref, w2BD_ref,
                 b2_ref, g2_ref, be2_ref, m2_ref, st_ref, acc_ref):
    pid = pl.program_id(0)

    @pl.when(pid == 0)
    def _():
        acc_ref[...] = jnp.zeros_like(acc_ref)

    m1 = _m1_of(tA_ref[...], tB_ref[...], ea_ref[...], weBD_ref[...], b1_ref)
    a1 = jnp.maximum(m1 * st1_ref[0:1, :] + st1_ref[1:2, :], 0.0)
    m2 = jnp.dot(a1, w2BD_ref[...], preferred_element_type=jnp.float32) \
        + b2_ref[0:1, :]
    m2 = jnp.where(_pmask(pid), m2, 0.0)
    m2_ref[...] = m2
    acc_ref[0:1, :] += jnp.sum(m2, axis=0, keepdims=True)
    acc_ref[1:2, :] += jnp.sum(m2 * m2, axis=0, keepdims=True)

    @pl.when(pid == GRID_E - 1)
    def _():
        _finish_stats(acc_ref, g2_ref, be2_ref, st_ref)


def _apply2_body(m2_ref, st2_ref, a2_ref):
    pid = pl.program_id(0)
    a2 = jnp.maximum(m2_ref[...] * st2_ref[0:1, :] + st2_ref[1:2, :], 0.0)
    a2_ref[...] = jnp.where(_pmask(pid), a2, 0.0)


def _oht(seg, pid):
    gids = lax.broadcasted_iota(jnp.int32, (C_PAD, GT), 1) + pid * GT
    valid = lax.broadcasted_iota(jnp.int32, (C_PAD, 1), 0) < C
    return (seg == gids) & valid


def _mx_body(hc_ref, seg_ref, lo_ref, lob_ref, lg_ref, mxc_ref, acc_ref):
    pid = pl.program_id(0)
    lg = jnp.dot(hc_ref[...], lo_ref[...],
                 preferred_element_type=jnp.float32)[:, 0:1] \
        + lob_ref[0:1, 0:1]

    @pl.when(pid == 0)
    def _():
        acc_ref[...] = jnp.zeros_like(acc_ref)
        lg_ref[...] = lg

    oht = _oht(seg_ref[...], pid)
    masked = jnp.where(oht, lg, -1e30)
    mxrow = jnp.max(masked, axis=0, keepdims=True)
    acc_ref[...] += jnp.sum(jnp.where(oht, mxrow, 0.0), axis=1, keepdims=True)

    @pl.when(pid == GRID_G - 1)
    def _():
        mxc_ref[...] = acc_ref[...]


def _lse_body(seg_ref, lg_ref, mxc_ref, out_ref, acc_ref):
    pid = pl.program_id(0)

    @pl.when(pid == 0)
    def _():
        acc_ref[...] = jnp.zeros_like(acc_ref)

    sh = lg_ref[...] - mxc_ref[...]
    valid = lax.broadcasted_iota(jnp.int32, (C_PAD, 1), 0) < C
    ex = jnp.where(valid, jnp.exp(sh), 0.0)
    oht = _oht(seg_ref[...], pid)
    srow = jnp.sum(jnp.where(oht, ex, 0.0), axis=0, keepdims=True)
    lserow = jnp.where(srow > 0.0, jnp.log(jnp.maximum(srow, 1e-37)), 0.0)
    acc_ref[...] += jnp.sum(jnp.where(oht, lserow, 0.0), axis=1, keepdims=True)

    @pl.when(pid == GRID_G - 1)
    def _():
        out_ref[...] = sh - acc_ref[...]


def kernel(x, edge_index, edge_attr, candidate_idxs, batch,
           lin_in_w, lin_in_b, w1, b1, w2, b2, g1, be1, g2, be2,
           lin_out_w, lin_out_b):
    f32 = jnp.float32
    eye4 = jnp.eye(4, dtype=f32)

    def bd(m32):
        return jnp.kron(eye4, m32)

    def t8(v):
        return jnp.broadcast_to(jnp.tile(v, 4).reshape(1, 128), (8, 128))

    src = edge_index[0]
    dst = edge_index[1]
    pad = E_PAD - E
    padidx = (jnp.arange(pad, dtype=jnp.int32) * 1031) % N
    dst_p = jnp.concatenate([dst, padidx])
    src_p = jnp.concatenate([src, padidx])
    dst3 = dst_p.reshape(16, NCHS, CH)
    t16 = jnp.kron(eye4, jnp.zeros((4, EMB), f32).at[:, 0:4].set(
        jnp.eye(4, dtype=f32)))
    colio = lax.broadcasted_iota(jnp.int32, (4, EAO, EAT), 2)
    rowio = 4 * lax.broadcasted_iota(jnp.int32, (4, EAO, EAT), 1) \
        + lax.broadcasted_iota(jnp.int32, (4, EAO, EAT), 0)
    sbig = (colio == rowio).astype(f32).reshape(4 * EAO, EAT)
    eaP = _tc_call(
        _eapack_body, (GRID_EA,),
        [pl.BlockSpec((EAT, 4), lambda i: (i, 0)),
         pl.BlockSpec((4 * EAO, EAT), lambda i: (0, 0)),
         pl.BlockSpec((16, 128), lambda i: (0, 0))],
        pl.BlockSpec((EAO, 128), lambda i: (i, 0)),
        jax.ShapeDtypeStruct((PR, 128), f32))(edge_attr, sbig, t16)

    cpadidx = (jnp.arange(C_PAD - C, dtype=jnp.int32) * 997) % N
    cand_p = jnp.concatenate([candidate_idxs, cpadidx])

    xP = jnp.zeros((N, EMB), f32).at[:, 0:2].set(x).reshape(NR, 128)
    lin32 = jnp.zeros((EMB, EMB), f32).at[0:2, :].set(lin_in_w.T)
    linBD = bd(lin32)
    b_in = t8(lin_in_b)

    wiBD = [bd(w1[l][:, 0:EMB].T) for l in range(L)]
    wjBD = [bd(w1[l][:, EMB:2 * EMB].T) for l in range(L)]
    weBD = [bd(jnp.zeros((EMB, EMB), f32).at[0:4, :].set(w1[l][:, 2 * EMB:].T))
            for l in range(L)]
    w2BD = [bd(w2[l].T) for l in range(L)]
    b1r = [t8(b1[l]) for l in range(L)]
    b2r = [t8(b2[l]) for l in range(L)]
    g1r = [t8(g1[l]) for l in range(L)]
    be1r = [t8(be1[l]) for l in range(L)]
    g2r = [t8(g2[l]) for l in range(L)]
    be2r = [t8(be2[l]) for l in range(L)]

    p128 = pl.BlockSpec((8, 128), lambda i: (0, 0))
    w128 = pl.BlockSpec((128, 128), lambda i: (0, 0))
    nspec = pl.BlockSpec((TNP, 128), lambda i: (i, 0))
    espec = pl.BlockSpec((TP, 128), lambda i: (i, 0))
    st_shape = jax.ShapeDtypeStruct((8, 128), f32)
    acc2 = pltpu.VMEM((8, 128), f32)

    hP = _tc_call(
        _h0_body, (1,),
        [pl.BlockSpec((NR, 128), lambda i: (0, 0)), w128, p128],
        pl.BlockSpec((NR, 128), lambda i: (0, 0)),
        jax.ShapeDtypeStruct((NR, 128), f32))(xP, linBD, b_in)

    aggP = jnp.zeros((NR, 128), f32)
    zrows = jnp.zeros((125, HEMB), f32)

    for l in range(L):
        hP, hiP, hjP = _tc_call(
            _dense_body, (GRID_N,),
            [nspec, nspec, w128, w128],
            [nspec, nspec, nspec],
            [jax.ShapeDtypeStruct((NR, 128), f32)] * 3,
        )(hP, aggP, wiBD[l], wjBD[l])

        tA, tB = _gather_edges(hiP.reshape(N, EMB), hjP.reshape(N, EMB),
                               dst_p, src_p)
        tAP = tA.reshape(PR, 128)
        tBP = tB.reshape(PR, 128)

        st1 = _tc_call(
            _stats1_body, (GRID_E,),
            [espec, espec, espec, w128, p128, p128, p128],
            p128, st_shape, [acc2],
        )(tAP, tBP, eaP, weBD[l], b1r[l], g1r[l], be1r[l])

        m2P, st2 = _tc_call(
            _stats2_body, (GRID_E,),
            [espec, espec, espec, w128, p128, p128, w128, p128, p128, p128],
            [espec, p128],
            [jax.ShapeDtypeStruct((PR, 128), f32), st_shape], [acc2],
        )(tAP, tBP, eaP, weBD[l], b1r[l], st1, w2BD[l], b2r[l], g2r[l],
          be2r[l])

        a2P = _tc_call(
            _apply2_body, (GRID_E,),
            [espec, p128], espec,
            jax.ShapeDtypeStruct((PR, 128), f32),
        )(m2P, st2)

        agg = _scatter_agg(a2P.reshape(E_PAD, EMB), dst3, zrows)
        aggP = agg.reshape(NR, 128)

    hfinP = _tc_call(
        _hfin_body, (GRID_N,),
        [nspec, nspec], nspec,
        jax.ShapeDtypeStruct((NR, 128), f32))(hP, aggP)

    hcb, segc = _gather_cands(hfinP.reshape(N, EMB), batch, cand_p)
    seg2 = segc.reshape(C_PAD, 1)

    lo8 = jnp.zeros((EMB, 8), f32).at[:, 0:1].set(lin_out_w.T)
    lob = jnp.full((8, 8), lin_out_b, f32)
    cspec = pl.BlockSpec((C_PAD, EMB), lambda i: (0, 0))
    s1spec = pl.BlockSpec((C_PAD, 1), lambda i: (0, 0))
    c1shape = jax.ShapeDtypeStruct((C_PAD, 1), f32)
    acc1 = pltpu.VMEM((C_PAD, 1), f32)

    lg, mxc = _tc_call(
        _mx_body, (GRID_G,),
        [cspec, s1spec, pl.BlockSpec((EMB, 8), lambda i: (0, 0)),
         pl.BlockSpec((8, 8), lambda i: (0, 0))],
        [s1spec, s1spec], [c1shape, c1shape], [acc1],
    )(hcb, seg2, lo8, lob)

    out = _tc_call(
        _lse_body, (GRID_G,),
        [s1spec, s1spec, s1spec], s1spec, c1shape, [acc1],
    )(seg2, lg, mxc)

    return out[:C, 0]
